# baseline (speedup 1.0000x reference)
import functools
import math

import jax
import jax.numpy as jnp
from jax import lax
from jax.experimental import pallas as pl
from jax.experimental.pallas import tpu as pltpu
from jax.experimental.pallas import tpu_sc as plsc

_LANES = 16
_IDX_CHUNK = 128


def _pool_sc(idx3, embedding, batch, ctx):
    d = embedding.shape[1]
    info = plsc.get_sparse_core_info()
    nw = info.num_cores * info.num_subcores
    n_chunks = idx3.shape[1]
    b_per_w = batch // nw
    g_per_w = n_chunks * _IDX_CHUNK

    mesh = plsc.VectorSubcoreMesh(core_axis_name="c", subcore_axis_name="s")

    @functools.partial(
        pl.kernel,
        mesh=mesh,
        out_type=jax.ShapeDtypeStruct((batch, d), jnp.float32),
        scratch_types=[
            pltpu.VMEM((n_chunks, _IDX_CHUNK), jnp.int32),
            pltpu.VMEM((g_per_w, d), jnp.float32),
            pltpu.VMEM((b_per_w, d), jnp.float32),
            pltpu.SemaphoreType.DMA,
        ],
        compiler_params=pltpu.CompilerParams(use_tc_tiling_on_sc=False),
    )
    def pool(idx_hbm, emb_hbm, out_hbm, idx_v, rows_v, acc_v, sem):
        wid = lax.axis_index("s") * info.num_cores + lax.axis_index("c")
        pltpu.sync_copy(idx_hbm.at[wid], idx_v)
        copies = [
            pltpu.async_copy(
                emb_hbm.at[idx_v.at[j]],
                rows_v.at[pl.ds(j * _IDX_CHUNK, _IDX_CHUNK)],
                sem,
            )
            for j in range(n_chunks)
        ]
        for cp in copies:
            cp.wait()

        def body(b, carry):
            g0 = b * ctx
            for dd in range(d // _LANES):
                sl = pl.ds(dd * _LANES, _LANES)
                s = rows_v[g0, sl]
                for c in range(1, ctx):
                    s = s + rows_v[g0 + c, sl]
                acc_v[b, sl] = s
            return carry

        lax.fori_loop(0, b_per_w, body, 0)
        pltpu.sync_copy(acc_v, out_hbm.at[pl.ds(wid * b_per_w, b_per_w)])

    return pool(idx3, embedding)


def _mm_body(x_ref, w_ref, b_ref, o_ref):
    o_ref[...] = (
        lax.dot_general(
            x_ref[...], w_ref[...],
            (((1,), (1,)), ((), ())),
            preferred_element_type=jnp.float32,
        )
        + b_ref[...]
    )


def _fc1_tc(x_sum, fc1_w, fc1_b, vb=2048):
    batch, d = x_sum.shape
    v = fc1_w.shape[0]
    nb = math.ceil(v / vb)
    return pl.pallas_call(
        _mm_body,
        grid=(nb,),
        in_specs=[
            pl.BlockSpec((batch, d), lambda j: (0, 0)),
            pl.BlockSpec((vb, d), lambda j: (j, 0)),
            pl.BlockSpec((1, vb), lambda j: (0, j)),
        ],
        out_specs=pl.BlockSpec((batch, vb), lambda j: (0, j)),
        out_shape=jax.ShapeDtypeStruct((batch, v), jnp.float32),
        compiler_params=pltpu.CompilerParams(
            dimension_semantics=("arbitrary",),
        ),
    )(x_sum, fc1_w, fc1_b.reshape(1, v))


def kernel(x_in, embedding, fc1_w, fc1_b):
    batch, ctx = x_in.shape
    info = plsc.get_sparse_core_info()
    nw = info.num_cores * info.num_subcores
    g_per_w = (batch // nw) * ctx
    n_chunks = g_per_w // _IDX_CHUNK
    idx3 = x_in.astype(jnp.int32).reshape(nw, n_chunks, _IDX_CHUNK)
    x_sum = _pool_sc(idx3, embedding, batch, ctx)
    return _fc1_tc(x_sum, fc1_w, fc1_b)
